# Initial kernel scaffold; baseline (speedup 1.0000x reference)
import functools

import jax
import jax.numpy as jnp
from jax import lax
from jax.experimental import pallas as pl
from jax.experimental.pallas import tpu as pltpu
from jax.experimental.pallas import tpu_sc as plsc

D = 16
SCALE = 4.0
CHUNK = 2048


def kernel(inp, emb_table):
    n_rows, n_cols = inp.shape
    B = n_rows * n_cols
    idx_flat = inp.reshape(B).astype(jnp.int32)

    info = plsc.get_sparse_core_info()
    nc, ns = info.num_cores, info.num_subcores
    nw = nc * ns
    per_w = B // nw
    n_chunks = per_w // CHUNK
    assert per_w * nw == B and n_chunks * CHUNK == per_w

    mesh = plsc.VectorSubcoreMesh(core_axis_name="c", subcore_axis_name="s")

    @functools.partial(
        pl.kernel,
        mesh=mesh,
        out_type=jax.ShapeDtypeStruct((B, D), jnp.float32),
        scratch_types=[
            pltpu.VMEM((CHUNK,), jnp.int32),
            pltpu.VMEM((CHUNK, D), jnp.float32),
            pltpu.SemaphoreType.DMA,
        ],
    )
    def sc_kernel(idx_hbm, table_hbm, out_hbm, idx_v, rows_v, sem):
        wid = lax.axis_index("s") * nc + lax.axis_index("c")
        base = wid * per_w

        def chunk_body(g, carry):
            off = base + g * CHUNK
            pltpu.sync_copy(idx_hbm.at[pl.ds(off, CHUNK)], idx_v)
            pltpu.async_copy(table_hbm.at[idx_v], rows_v, sem).wait()

            def scale_body(i, c):
                rows_v[i] = rows_v[i] * SCALE
                return c

            lax.fori_loop(0, CHUNK, scale_body, 0, unroll=8)
            pltpu.sync_copy(rows_v, out_hbm.at[pl.ds(off, CHUNK)])
            return carry

        lax.fori_loop(0, n_chunks, chunk_body, 0)

    out = sc_kernel(idx_flat, emb_table)
    return out.reshape(n_rows, n_cols, D)


# speedup vs baseline: 2.4176x; 2.4176x over previous
import functools

import jax
import jax.numpy as jnp
from jax import lax
from jax.experimental import pallas as pl
from jax.experimental.pallas import tpu as pltpu
from jax.experimental.pallas import tpu_sc as plsc

D = 16
SCALE = 4.0
CHUNK = 2048


def kernel(inp, emb_table):
    n_rows, n_cols = inp.shape
    B = n_rows * n_cols
    idx_flat = inp.reshape(B).astype(jnp.int32)

    info = plsc.get_sparse_core_info()
    nc, ns = info.num_cores, info.num_subcores
    nw = nc * ns
    per_w = B // nw
    n_chunks = per_w // CHUNK
    assert per_w * nw == B and n_chunks * CHUNK == per_w

    mesh = plsc.VectorSubcoreMesh(core_axis_name="c", subcore_axis_name="s")

    @functools.partial(
        pl.kernel,
        mesh=mesh,
        out_type=jax.ShapeDtypeStruct((B, D), jnp.float32),
        scratch_types=[
            pltpu.VMEM((CHUNK,), jnp.int32),
            pltpu.VMEM((CHUNK, D), jnp.float32),
            pltpu.SemaphoreType.DMA,
        ],
        compiler_params=pltpu.CompilerParams(use_tc_tiling_on_sc=False),
    )
    def sc_kernel(idx_hbm, table_hbm, out_hbm, idx_v, rows_v, sem):
        wid = lax.axis_index("s") * nc + lax.axis_index("c")
        base = wid * per_w

        def chunk_body(g, carry):
            off = base + g * CHUNK
            pltpu.sync_copy(idx_hbm.at[pl.ds(off, CHUNK)], idx_v)
            pltpu.async_copy(table_hbm.at[idx_v], rows_v, sem).wait()

            def scale_body(i, c):
                rows_v[i] = rows_v[i] * SCALE
                return c

            lax.fori_loop(0, CHUNK, scale_body, 0, unroll=8)
            pltpu.sync_copy(rows_v, out_hbm.at[pl.ds(off, CHUNK)])
            return carry

        lax.fori_loop(0, n_chunks, chunk_body, 0)

    out = sc_kernel(idx_flat, emb_table)
    return out.reshape(n_rows, n_cols, D)


# speedup vs baseline: 2.5561x; 1.0573x over previous
import functools

import jax
import jax.numpy as jnp
from jax import lax
from jax.experimental import pallas as pl
from jax.experimental.pallas import tpu as pltpu
from jax.experimental.pallas import tpu_sc as plsc

D = 16
SCALE = 4.0
CHUNK = 1024
NBUF = 4


def kernel(inp, emb_table):
    n_rows, n_cols = inp.shape
    B = n_rows * n_cols
    idx_flat = inp.reshape(B).astype(jnp.int32)

    info = plsc.get_sparse_core_info()
    nc, ns = info.num_cores, info.num_subcores
    nw = nc * ns
    per_w = B // nw
    n_chunks = per_w // CHUNK
    assert per_w * nw == B and n_chunks * CHUNK == per_w
    assert n_chunks % NBUF == 0

    mesh = plsc.VectorSubcoreMesh(core_axis_name="c", subcore_axis_name="s")

    @functools.partial(
        pl.kernel,
        mesh=mesh,
        out_type=jax.ShapeDtypeStruct((B, D), jnp.float32),
        scratch_types=(
            [pltpu.VMEM((CHUNK,), jnp.int32) for _ in range(NBUF)]
            + [pltpu.VMEM((CHUNK, D), jnp.float32) for _ in range(NBUF)]
            + [pltpu.SemaphoreType.DMA for _ in range(3 * NBUF)]
        ),
        compiler_params=pltpu.CompilerParams(use_tc_tiling_on_sc=False),
    )
    def sc_kernel(idx_hbm, table_hbm, out_hbm, *refs):
        idxb = refs[0:NBUF]
        rowsb = refs[NBUF:2 * NBUF]
        isem = refs[2 * NBUF:3 * NBUF]
        gsem = refs[3 * NBUF:4 * NBUF]
        osem = refs[4 * NBUF:5 * NBUF]

        wid = lax.axis_index("s") * nc + lax.axis_index("c")
        base = wid * per_w

        def idx_copy(g, b):
            return pltpu.make_async_copy(
                idx_hbm.at[pl.ds(base + g * CHUNK, CHUNK)], idxb[b], isem[b])

        def gather_copy(b):
            return pltpu.make_async_copy(table_hbm.at[idxb[b]], rowsb[b], gsem[b])

        def store_copy(g, b):
            return pltpu.make_async_copy(
                rowsb[b], out_hbm.at[pl.ds(base + g * CHUNK, CHUNK)], osem[b])

        for c in range(NBUF - 1):
            idx_copy(c, c).start()
        for c in range(NBUF - 2):
            idx_copy(c, c).wait()
            gather_copy(c).start()

        def outer(ko, carry):
            for b in range(NBUF):
                g = ko * NBUF + b

                h1 = g + (NBUF - 1)
                b1 = (b + NBUF - 1) % NBUF

                @pl.when(h1 < n_chunks)
                def _():
                    idx_copy(h1, b1).start()

                h2 = g + (NBUF - 2)
                b2 = (b + NBUF - 2) % NBUF

                @pl.when(jnp.logical_and(h2 < n_chunks, h2 >= NBUF))
                def _():
                    store_copy(h2 - NBUF, b2).wait()

                @pl.when(h2 < n_chunks)
                def _():
                    idx_copy(h2, b2).wait()
                    gather_copy(b2).start()

                gather_copy(b).wait()
                rb = rowsb[b]

                @plsc.parallel_loop(0, CHUNK, 1, unroll=8)
                def _(i):
                    rb[i] = rb[i] * SCALE

                store_copy(g, b).start()
            return carry

        lax.fori_loop(0, n_chunks // NBUF, outer, 0)

        for b in range(NBUF):
            store_copy(n_chunks - NBUF + b, b).wait()

    out = sc_kernel(idx_flat, emb_table)
    return out.reshape(n_rows, n_cols, D)


# speedup vs baseline: 3.8553x; 1.5083x over previous
import functools

import jax
import jax.numpy as jnp
from jax import lax
from jax.experimental import pallas as pl
from jax.experimental.pallas import tpu as pltpu
from jax.experimental.pallas import tpu_sc as plsc

D = 16
SCALE = 4.0
CHUNK = 512
NBUF = 5
JT, IT = 8, 128


def kernel(inp, emb_table):
    n_i, n_j = inp.shape
    n_vocab = emb_table.shape[0]
    B = n_i * n_j
    jb, ib = n_j // JT, n_i // IT
    assert jb * JT == n_j and ib * IT == n_i

    idx_lin = (
        jnp.transpose(inp)
        .reshape(jb, JT, ib, IT)
        .transpose(0, 2, 1, 3)
        .reshape(B)
        .astype(jnp.int32)
    )

    info = plsc.get_sparse_core_info()
    nc, ns = info.num_cores, info.num_subcores
    nw = nc * ns
    per_w = B // (nw * CHUNK)
    assert per_w * nw * CHUNK == B and per_w % NBUF == 0
    JH = (IT * CHUNK) // (IT * IT)

    mesh = plsc.VectorSubcoreMesh(core_axis_name="c", subcore_axis_name="s")

    @functools.partial(
        pl.kernel,
        mesh=mesh,
        out_type=jax.ShapeDtypeStruct((n_j, D // JT, ib, JT, IT), jnp.float32),
        scratch_types=(
            [pltpu.VMEM((CHUNK,), jnp.int32) for _ in range(NBUF)]
            + [pltpu.VMEM((CHUNK, D), jnp.float32) for _ in range(NBUF)]
            + [pltpu.VMEM((JH, D // JT, JT, IT), jnp.float32) for _ in range(NBUF)]
            + [pltpu.SemaphoreType.DMA for _ in range(3 * NBUF)]
        ),
        compiler_params=pltpu.CompilerParams(
            use_tc_tiling_on_sc=False, needs_layout_passes=False),
    )
    def sc_kernel(idx_hbm, table_hbm, out_hbm, *refs):
        idxb = refs[0:NBUF]
        rowsb = refs[NBUF:2 * NBUF]
        obufb = refs[2 * NBUF:3 * NBUF]
        isem = refs[3 * NBUF:4 * NBUF]
        gsem = refs[4 * NBUF:5 * NBUF]
        osem = refs[5 * NBUF:6 * NBUF]

        wid = lax.axis_index("s") * nc + lax.axis_index("c")
        g0 = wid * per_w

        lane = lax.iota(jnp.int32, D)
        vdhi = lane >> 3
        vdlo = lane & 7

        def idx_copy(g, b):
            return pltpu.make_async_copy(
                idx_hbm.at[pl.ds((g0 + g) * CHUNK, CHUNK)], idxb[b], isem[b])

        def gather_copy(b):
            return pltpu.make_async_copy(table_hbm.at[idxb[b]], rowsb[b], gsem[b])

        def store_copy(g, b):
            G = g0 + g
            j0 = ((G >> 8) << 3) + ((G & 1) << 2)
            ihi = (G >> 1) & (ib - 1)
            return pltpu.make_async_copy(
                obufb[b], out_hbm.at[pl.ds(j0, JH), :, ihi, :, :], osem[b])

        for c in range(NBUF - 1):
            idx_copy(c, c).start()
        for c in range(NBUF - 2):
            idx_copy(c, c).wait()
            gather_copy(c).start()

        def outer(ko, carry):
            for b in range(NBUF):
                g = ko * NBUF + b

                h1 = g + (NBUF - 1)
                b1 = (b + NBUF - 1) % NBUF

                @pl.when(h1 < per_w)
                def _():
                    idx_copy(h1, b1).start()

                h2 = g + (NBUF - 2)
                b2 = (b + NBUF - 2) % NBUF

                @pl.when(jnp.logical_and(h2 < per_w, h2 >= NBUF))
                def _():
                    store_copy(h2 - NBUF, b2).wait()

                @pl.when(h2 < per_w)
                def _():
                    idx_copy(h2, b2).wait()
                    gather_copy(b2).start()

                gather_copy(b).wait()
                rb = rowsb[b]
                ob = obufb[b]

                def xpose(p, c):
                    jlo = p >> 7
                    ilo = p & (IT - 1)
                    x = rb[p] * SCALE
                    plsc.store_scatter(
                        ob,
                        [jnp.broadcast_to(jlo, (D,)), vdhi, vdlo,
                         jnp.broadcast_to(ilo, (D,))],
                        x)
                    return c

                lax.fori_loop(0, CHUNK, xpose, 0, unroll=4)

                store_copy(g, b).start()
            return carry

        lax.fori_loop(0, per_w // NBUF, outer, 0)

        for b in range(NBUF):
            store_copy(per_w - NBUF + b, b).wait()

    out5d = sc_kernel(idx_lin, emb_table)
    return out5d.transpose(2, 4, 0, 1, 3).reshape(n_i, n_j, D)


# speedup vs baseline: 5.0891x; 1.3200x over previous
import functools

import jax
import jax.numpy as jnp
from jax import lax
from jax.experimental import pallas as pl
from jax.experimental.pallas import tpu as pltpu
from jax.experimental.pallas import tpu_sc as plsc

D = 16
SCALE = 4.0
CHUNK = 512
NBUF = 5
JT, IT = 8, 128


def kernel(inp, emb_table):
    n_i, n_j = inp.shape
    n_vocab = emb_table.shape[0]
    B = n_i * n_j
    jb, ib = n_j // JT, n_i // IT
    assert jb * JT == n_j and ib * IT == n_i

    idx_lin = (
        jnp.transpose(inp)
        .reshape(jb, JT, ib, IT)
        .transpose(0, 2, 1, 3)
        .reshape(B)
        .astype(jnp.int32)
    )

    info = plsc.get_sparse_core_info()
    nc, ns = info.num_cores, info.num_subcores
    nw = nc * ns
    per_w = B // (nw * CHUNK)
    assert per_w * nw * CHUNK == B and per_w % NBUF == 0
    JH = (IT * CHUNK) // (IT * IT)

    mesh = plsc.VectorSubcoreMesh(core_axis_name="c", subcore_axis_name="s")

    @functools.partial(
        pl.kernel,
        mesh=mesh,
        out_type=jax.ShapeDtypeStruct((n_j, D // JT, ib, JT, IT), jnp.float32),
        scratch_types=(
            [pltpu.VMEM((CHUNK,), jnp.int32) for _ in range(NBUF)]
            + [pltpu.VMEM((CHUNK, D), jnp.float32) for _ in range(NBUF)]
            + [pltpu.VMEM((JH, D // JT, JT, IT), jnp.float32) for _ in range(NBUF)]
            + [pltpu.SemaphoreType.DMA for _ in range(3 * NBUF)]
        ),
        compiler_params=pltpu.CompilerParams(
            use_tc_tiling_on_sc=False, needs_layout_passes=False),
    )
    def sc_kernel(idx_hbm, table_hbm, out_hbm, *refs):
        idxb = refs[0:NBUF]
        rowsb = refs[NBUF:2 * NBUF]
        obufb = refs[2 * NBUF:3 * NBUF]
        isem = refs[3 * NBUF:4 * NBUF]
        gsem = refs[4 * NBUF:5 * NBUF]
        osem = refs[5 * NBUF:6 * NBUF]

        wid = lax.axis_index("s") * nc + lax.axis_index("c")
        g0 = wid * per_w

        lane = lax.iota(jnp.int32, D)
        vdhi = lane >> 3
        vdlo = lane & 7

        def idx_copy(g, b):
            return pltpu.make_async_copy(
                idx_hbm.at[pl.ds((g0 + g) * CHUNK, CHUNK)], idxb[b], isem[b])

        def gather_copy(b):
            return pltpu.make_async_copy(table_hbm.at[idxb[b]], rowsb[b], gsem[b])

        def store_copies(g, b):
            G = g0 + g
            j0 = ((G >> 8) << 3) + ((G & 1) << 2)
            ihi = (G >> 1) & (ib - 1)
            return [
                pltpu.make_async_copy(
                    obufb[b].at[jl, dh],
                    out_hbm.at[j0 + jl, dh, ihi, :, :], osem[b])
                for jl in range(JH) for dh in range(D // JT)
            ]

        for c in range(NBUF - 1):
            idx_copy(c, c).start()
        for c in range(NBUF - 2):
            idx_copy(c, c).wait()
            gather_copy(c).start()

        def outer(ko, carry):
            for b in range(NBUF):
                g = ko * NBUF + b

                h1 = g + (NBUF - 1)
                b1 = (b + NBUF - 1) % NBUF

                @pl.when(h1 < per_w)
                def _():
                    idx_copy(h1, b1).start()

                h2 = g + (NBUF - 2)
                b2 = (b + NBUF - 2) % NBUF

                @pl.when(jnp.logical_and(h2 < per_w, h2 >= NBUF))
                def _():
                    for c in store_copies(h2 - NBUF, b2):
                        c.wait()

                @pl.when(h2 < per_w)
                def _():
                    idx_copy(h2, b2).wait()
                    gather_copy(b2).start()

                gather_copy(b).wait()
                rb = rowsb[b]
                ob = obufb[b]

                @plsc.parallel_loop(0, CHUNK, 1, unroll=4)
                def _(p):
                    jlo = p >> 7
                    ilo = p & (IT - 1)
                    x = rb[p] * SCALE
                    plsc.store_scatter(
                        ob,
                        [jnp.broadcast_to(jlo, (D,)), vdhi, vdlo,
                         jnp.broadcast_to(ilo, (D,))],
                        x)

                for c in store_copies(g, b):
                    c.start()
            return carry

        lax.fori_loop(0, per_w // NBUF, outer, 0)

        for b in range(NBUF):
            for c in store_copies(per_w - NBUF + b, b):
                c.wait()

    out5d = sc_kernel(idx_lin, emb_table)
    return out5d.transpose(2, 4, 0, 1, 3).reshape(n_i, n_j, D)


# speedup vs baseline: 5.2056x; 1.0229x over previous
import functools

import jax
import jax.numpy as jnp
from jax import lax
from jax.experimental import pallas as pl
from jax.experimental.pallas import tpu as pltpu
from jax.experimental.pallas import tpu_sc as plsc

D = 16
SCALE = 4.0
CHUNK = 512
NBUF = 5
JT, IT = 8, 128
RUN = JT * IT


def kernel(inp, emb_table):
    n_i, n_j = inp.shape
    B = n_i * n_j
    jb, ib = n_j // JT, n_i // IT
    assert jb * JT == n_j and ib * IT == n_i

    idx_lin = (
        jnp.transpose(inp)
        .reshape(jb, JT, ib, IT)
        .transpose(0, 2, 1, 3)
        .reshape(B)
        .astype(jnp.int32)
    )

    info = plsc.get_sparse_core_info()
    nc, ns = info.num_cores, info.num_subcores
    nw = nc * ns
    per_w = B // (nw * CHUNK)
    assert per_w * nw * CHUNK == B and per_w % NBUF == 0
    JH = CHUNK // IT
    NRUN = JH * D // JT

    mesh = plsc.VectorSubcoreMesh(core_axis_name="c", subcore_axis_name="s")

    @functools.partial(
        pl.kernel,
        mesh=mesh,
        out_type=jax.ShapeDtypeStruct((B * D,), jnp.float32),
        scratch_types=(
            [pltpu.VMEM((CHUNK,), jnp.int32) for _ in range(NBUF)]
            + [pltpu.VMEM((CHUNK, D), jnp.float32) for _ in range(NBUF)]
            + [pltpu.VMEM((CHUNK * D,), jnp.float32) for _ in range(NBUF)]
            + [pltpu.SemaphoreType.DMA for _ in range(3 * NBUF)]
        ),
        compiler_params=pltpu.CompilerParams(
            use_tc_tiling_on_sc=False, needs_layout_passes=False),
    )
    def sc_kernel(idx_hbm, table_hbm, out_hbm, *refs):
        idxb = refs[0:NBUF]
        rowsb = refs[NBUF:2 * NBUF]
        obufb = refs[2 * NBUF:3 * NBUF]
        isem = refs[3 * NBUF:4 * NBUF]
        gsem = refs[4 * NBUF:5 * NBUF]
        osem = refs[5 * NBUF:6 * NBUF]

        wid = lax.axis_index("s") * nc + lax.axis_index("c")
        g0 = wid * per_w

        lane = lax.iota(jnp.int32, D)
        vconst = ((lane >> 3) << 10) + ((lane & 7) << 7)

        def idx_copy(g, b):
            return pltpu.make_async_copy(
                idx_hbm.at[pl.ds((g0 + g) * CHUNK, CHUNK)], idxb[b], isem[b])

        def gather_copy(b):
            return pltpu.make_async_copy(table_hbm.at[idxb[b]], rowsb[b], gsem[b])

        def store_copies(g, b):
            G = g0 + g
            j0 = ((G >> 8) << 3) + ((G & 1) << 2)
            ihi = (G >> 1) & (ib - 1)
            out = []
            for r in range(NRUN):
                jl, dh = r >> 1, r & 1
                base = (((((j0 + jl) << 1) + dh) * ib + ihi) << 10)
                base = pl.multiple_of(base, RUN)
                out.append(pltpu.make_async_copy(
                    obufb[b].at[pl.ds(r * RUN, RUN)],
                    out_hbm.at[pl.ds(base, RUN)], osem[b]))
            return out

        for c in range(NBUF - 1):
            idx_copy(c, c).start()
        for c in range(NBUF - 2):
            idx_copy(c, c).wait()
            gather_copy(c).start()

        def outer(ko, carry):
            for b in range(NBUF):
                g = ko * NBUF + b

                h1 = g + (NBUF - 1)
                b1 = (b + NBUF - 1) % NBUF

                @pl.when(h1 < per_w)
                def _():
                    idx_copy(h1, b1).start()

                h2 = g + (NBUF - 2)
                b2 = (b + NBUF - 2) % NBUF

                @pl.when(jnp.logical_and(h2 < per_w, h2 >= NBUF))
                def _():
                    for c in store_copies(h2 - NBUF, b2):
                        c.wait()

                @pl.when(h2 < per_w)
                def _():
                    idx_copy(h2, b2).wait()
                    gather_copy(b2).start()

                gather_copy(b).wait()
                rb = rowsb[b]
                ob = obufb[b]

                @plsc.parallel_loop(0, CHUNK, 1, unroll=8)
                def _(p):
                    s = ((p >> 7) << 11) + (p & (IT - 1))
                    x = rb[p] * SCALE
                    plsc.store_scatter(ob, [vconst + s], x)

                for c in store_copies(g, b):
                    c.start()
            return carry

        lax.fori_loop(0, per_w // NBUF, outer, 0)

        for b in range(NBUF):
            for c in store_copies(per_w - NBUF + b, b):
                c.wait()

    out_flat = sc_kernel(idx_lin, emb_table)
    return (out_flat.reshape(n_j, D // JT, ib, JT, IT)
            .transpose(2, 4, 0, 1, 3).reshape(n_i, n_j, D))


# speedup vs baseline: 9.5424x; 1.8331x over previous
import functools

import jax
import jax.numpy as jnp
from jax import lax
from jax.experimental import pallas as pl
from jax.experimental.pallas import tpu as pltpu
from jax.experimental.pallas import tpu_sc as plsc

D = 16
SCALE = 4.0
CHUNK = 512
NBUF = 5
JT, IT = 8, 128
RUN = JT * IT


def kernel(inp, emb_table):
    n_i, n_j = inp.shape
    B = n_i * n_j
    jb, ib = n_j // JT, n_i // IT
    assert jb * JT == n_j and ib * IT == n_i

    idx_lin = (
        jnp.transpose(inp)
        .reshape(jb, JT, ib, IT)
        .transpose(0, 2, 1, 3)
        .reshape(B)
        .astype(jnp.int32)
    )

    info = plsc.get_sparse_core_info()
    nc, ns = info.num_cores, info.num_subcores
    nw = nc * ns
    per_w = B // (nw * CHUNK)
    assert per_w * nw * CHUNK == B and per_w % NBUF == 0
    JH = CHUNK // IT
    NRUN = JH * D // JT

    mesh = plsc.VectorSubcoreMesh(core_axis_name="c", subcore_axis_name="s")

    @functools.partial(
        pl.kernel,
        mesh=mesh,
        out_type=jax.ShapeDtypeStruct((B * D,), jnp.float32),
        scratch_types=(
            [pltpu.VMEM((CHUNK,), jnp.int32) for _ in range(NBUF)]
            + [pltpu.VMEM((CHUNK, D), jnp.float32) for _ in range(NBUF)]
            + [pltpu.VMEM((CHUNK * D,), jnp.float32) for _ in range(NBUF)]
            + [pltpu.SemaphoreType.DMA for _ in range(3 * NBUF)]
        ),
        compiler_params=pltpu.CompilerParams(
            use_tc_tiling_on_sc=False, needs_layout_passes=False),
    )
    def sc_kernel(idx_hbm, table_hbm, out_hbm, *refs):
        idxb = refs[0:NBUF]
        rowsb = refs[NBUF:2 * NBUF]
        obufb = refs[2 * NBUF:3 * NBUF]
        isem = refs[3 * NBUF:4 * NBUF]
        gsem = refs[4 * NBUF:5 * NBUF]
        osem = refs[5 * NBUF:6 * NBUF]

        wid = lax.axis_index("s") * nc + lax.axis_index("c")
        g0 = wid * per_w

        lane = lax.iota(jnp.int32, D)

        def idx_copy(g, b):
            return pltpu.make_async_copy(
                idx_hbm.at[pl.ds((g0 + g) * CHUNK, CHUNK)], idxb[b], isem[b])

        def gather_copy(b):
            return pltpu.make_async_copy(table_hbm.at[idxb[b]], rowsb[b], gsem[b])

        def store_copies(g, b):
            G = g0 + g
            j0 = ((G >> 8) << 3) + ((G & 1) << 2)
            ihi = (G >> 1) & (ib - 1)
            out = []
            for r in range(NRUN):
                jl, dh = r >> 1, r & 1
                base = (((((j0 + jl) << 1) + dh) * ib + ihi) << 10)
                base = pl.multiple_of(base, RUN)
                out.append(pltpu.make_async_copy(
                    obufb[b].at[pl.ds(r * RUN, RUN)],
                    out_hbm.at[pl.ds(base, RUN)], osem[b]))
            return out

        for c in range(NBUF - 1):
            idx_copy(c, c).start()
        for c in range(NBUF - 2):
            idx_copy(c, c).wait()
            gather_copy(c).start()

        def outer(ko, carry):
            for b in range(NBUF):
                g = ko * NBUF + b

                h1 = g + (NBUF - 1)
                b1 = (b + NBUF - 1) % NBUF

                @pl.when(h1 < per_w)
                def _():
                    idx_copy(h1, b1).start()

                h2 = g + (NBUF - 2)
                b2 = (b + NBUF - 2) % NBUF

                @pl.when(jnp.logical_and(h2 < per_w, h2 >= NBUF))
                def _():
                    for c in store_copies(h2 - NBUF, b2):
                        c.wait()

                @pl.when(h2 < per_w)
                def _():
                    idx_copy(h2, b2).wait()
                    gather_copy(b2).start()

                gather_copy(b).wait()
                rb = rowsb[b]
                ob = obufb[b]

                @plsc.parallel_loop(0, CHUNK, 1, unroll=8)
                def _(p):
                    perm = (lane - p) & (D - 1)
                    x = (rb[p] * SCALE).at[perm].get(mode="promise_in_bounds")
                    rb[p] = x

                @plsc.parallel_loop(0, CHUNK, 1, unroll=8)
                def _(q):
                    jlo = q >> 7
                    d = (q >> 3) & (D - 1)
                    ilo0 = (q & 7) << 4
                    p0 = (jlo << 7) + ilo0
                    vp = p0 + lane
                    vd = (d + vp) & (D - 1)
                    v = plsc.load_gather(rb, [vp, vd])
                    off = (jlo << 11) + ((d >> 3) << 10) + ((d & 7) << 7) + ilo0
                    ob[pl.ds(pl.multiple_of(off, D), D)] = v

                for c in store_copies(g, b):
                    c.start()
            return carry

        lax.fori_loop(0, per_w // NBUF, outer, 0)

        for b in range(NBUF):
            for c in store_copies(per_w - NBUF + b, b):
                c.wait()

    out_flat = sc_kernel(idx_lin, emb_table)
    return (out_flat.reshape(n_j, D // JT, ib, JT, IT)
            .transpose(2, 4, 0, 1, 3).reshape(n_i, n_j, D))
